# baseline (speedup 1.0000x reference)
import functools

import jax
import jax.numpy as jnp
from jax import lax
from jax.experimental import pallas as pl
from jax.experimental.pallas import tpu as pltpu
from jax.experimental.pallas import tpu_sc as plsc

B, S, D = 4, 4096, 2048
N = 128
TOTAL = B * N
L = 16
HALF = TOTAL // 2
NS = 16
BPW = HALF // NS
CHUNKS = D // L
UNROLL = 8

_mesh = plsc.VectorSubcoreMesh(
    core_axis_name="c", subcore_axis_name="s", num_cores=1)


@functools.partial(
    pl.kernel,
    mesh=_mesh,
    out_type=jax.ShapeDtypeStruct((HALF, D), jnp.float32),
    scratch_types=[
        pltpu.VMEM((BPW,), jnp.int32),
        pltpu.VMEM((BPW,), jnp.int32),
        pltpu.VMEM((BPW, D), jnp.float32),
        pltpu.SemaphoreType.DMA,
    ],
)
def _sc_gather(wv_hbm, ids_hbm, mask_hbm, out_hbm, idx_v, mask_v, rows_v, sem):
    base = lax.axis_index("s") * BPW
    pltpu.sync_copy(ids_hbm.at[pl.ds(base, BPW)], idx_v)
    pltpu.sync_copy(mask_hbm.at[pl.ds(base, BPW)], mask_v)
    boff = ((base + HALF) // N) * S
    idx_v[...] = idx_v[...] + boff
    pltpu.async_copy(wv_hbm.at[idx_v], rows_v, sem).wait()

    zero = jnp.zeros((L,), jnp.float32)
    mask_reg = mask_v[...]

    for r in range(BPW):
        @pl.when(mask_reg[r] == 0)
        def _zero_row(r=r):
            def col_body(j, _):
                for u in range(UNROLL):
                    rows_v[r, pl.ds((j * UNROLL + u) * L, L)] = zero
                return 0

            lax.fori_loop(0, CHUNKS // UNROLL, col_body, 0)

    pltpu.sync_copy(rows_v, out_hbm.at[pl.ds(base, BPW)])


def _tc_body(ids_ref, mask_ref, wv_ref, out_ref):
    i = pl.program_id(0)
    out_ref[...] = wv_ref[...] * mask_ref[i].astype(jnp.float32)


_tc_gather = pl.pallas_call(
    _tc_body,
    grid_spec=pltpu.PrefetchScalarGridSpec(
        num_scalar_prefetch=2,
        grid=(HALF,),
        in_specs=[
            pl.BlockSpec(
                (1, 1, D),
                lambda i, ids, msk: (ids[i] + lax.div(i, N) * S, 0, 0),
            ),
        ],
        out_specs=pl.BlockSpec((1, 1, D), lambda i, ids, msk: (i, 0, 0)),
    ),
    out_shape=jax.ShapeDtypeStruct((HALF, 1, D), jnp.float32),
)


def kernel(word_vectors, sent_rep_token_ids, sent_rep_mask):
    wv2d = word_vectors.reshape(B * S, D)
    ids = sent_rep_token_ids.reshape(TOTAL)
    msk = sent_rep_mask.reshape(TOTAL)
    wv3d = word_vectors.reshape(B * S, 1, D)
    lo = _tc_gather(ids[:HALF], msk[:HALF], wv3d).reshape(HALF, D)
    hi = _sc_gather(wv2d, ids[HALF:], msk[HALF:])
    out = jnp.concatenate([lo, hi], axis=0)
    return out.reshape(B, N, D), sent_rep_mask


# speedup vs baseline: 22.5107x; 22.5107x over previous
import functools

import jax
import jax.numpy as jnp
from jax import lax
from jax.experimental import pallas as pl
from jax.experimental.pallas import tpu as pltpu
from jax.experimental.pallas import tpu_sc as plsc

B, S, D = 4, 4096, 2048
N = 128
TOTAL = B * N
L = 16
NC, NS = 2, 16
NW = NC * NS
BPW = TOTAL // NW
HB = BPW // 2
CHUNKS = D // L
UNROLL = 8

_mesh = plsc.VectorSubcoreMesh(core_axis_name="c", subcore_axis_name="s")


@functools.partial(
    pl.kernel,
    mesh=_mesh,
    out_type=jax.ShapeDtypeStruct((TOTAL, D), jnp.float32),
    scratch_types=[
        pltpu.VMEM((2 * BPW,), jnp.int32),
        pltpu.VMEM((BPW, D), jnp.float32),
        pltpu.SemaphoreType.DMA,
        pltpu.SemaphoreType.DMA,
        pltpu.SemaphoreType.DMA,
    ],
)
def _gather_pool(wv_hbm, im_hbm, out_hbm, im_v, rows_v, sem0, sem1, semw):
    wid = lax.axis_index("s") * NC + lax.axis_index("c")
    base = wid * BPW
    pltpu.sync_copy(im_hbm.at[pl.ds(wid * 2 * BPW, 2 * BPW)], im_v)
    boff = (base // N) * S
    im_v[pl.ds(0, BPW)] = im_v[pl.ds(0, BPW)] + boff

    g0 = pltpu.async_copy(wv_hbm.at[im_v.at[pl.ds(0, HB)]],
                          rows_v.at[pl.ds(0, HB)], sem0)
    g1 = pltpu.async_copy(wv_hbm.at[im_v.at[pl.ds(HB, HB)]],
                          rows_v.at[pl.ds(HB, HB)], sem1)

    zero = jnp.zeros((L,), jnp.float32)
    mask_reg = im_v[pl.ds(BPW, BPW)]

    def zero_masked(r0):
        for r in range(r0, r0 + HB):
            @pl.when(mask_reg[r] == 0)
            def _zero_row(r=r):
                def col_body(j, _):
                    for u in range(UNROLL):
                        rows_v[r, pl.ds((j * UNROLL + u) * L, L)] = zero
                    return 0

                lax.fori_loop(0, CHUNKS // UNROLL, col_body, 0)

    g0.wait()
    zero_masked(0)
    w0 = pltpu.async_copy(rows_v.at[pl.ds(0, HB)],
                          out_hbm.at[pl.ds(base, HB)], semw)
    g1.wait()
    zero_masked(HB)
    w1 = pltpu.async_copy(rows_v.at[pl.ds(HB, HB)],
                          out_hbm.at[pl.ds(base + HB, HB)], semw)
    w0.wait()
    w1.wait()


def kernel(word_vectors, sent_rep_token_ids, sent_rep_mask):
    wv2d = word_vectors.reshape(B * S, D)
    im = jnp.stack([sent_rep_token_ids.reshape(NW, BPW),
                    sent_rep_mask.reshape(NW, BPW)], axis=1).reshape(-1)
    out = _gather_pool(wv2d, im)
    return out.reshape(B, N, D), sent_rep_mask


# speedup vs baseline: 22.5424x; 1.0014x over previous
import functools

import jax
import jax.numpy as jnp
from jax import lax
from jax.experimental import pallas as pl
from jax.experimental.pallas import tpu as pltpu
from jax.experimental.pallas import tpu_sc as plsc

B, S, D = 4, 4096, 2048
N = 128
TOTAL = B * N
L = 16
NC, NS = 2, 16
NW = NC * NS
BPW = TOTAL // NW
NSTAGE = 4
HB = BPW // NSTAGE
IDS_PAD = 8
WBLK = NSTAGE * IDS_PAD + BPW
CHUNKS = D // L
UNROLL = 8

_mesh = plsc.VectorSubcoreMesh(core_axis_name="c", subcore_axis_name="s")


@functools.partial(
    pl.kernel,
    mesh=_mesh,
    out_type=jax.ShapeDtypeStruct((TOTAL, D), jnp.float32),
    scratch_types=[
        pltpu.VMEM((WBLK,), jnp.int32),
        pltpu.VMEM((HB, D), jnp.float32),
        pltpu.VMEM((HB, D), jnp.float32),
        pltpu.VMEM((HB, D), jnp.float32),
        pltpu.VMEM((HB, D), jnp.float32),
        pltpu.SemaphoreType.DMA,
        pltpu.SemaphoreType.DMA,
        pltpu.SemaphoreType.DMA,
        pltpu.SemaphoreType.DMA,
        pltpu.SemaphoreType.DMA,
    ],
)
def _gather_pool(wv_hbm, im_hbm, out_hbm, im_v, rows0, rows1, rows2, rows3,
                 sem0, sem1, sem2, sem3, semw):
    wid = lax.axis_index("s") * NC + lax.axis_index("c")
    base = wid * BPW
    pltpu.sync_copy(im_hbm.at[pl.ds(wid * WBLK, WBLK)], im_v)
    boff = (base // N) * S
    im_v[pl.ds(0, L)] = im_v[pl.ds(0, L)] + boff
    im_v[pl.ds(L, L)] = im_v[pl.ds(L, L)] + boff

    sems = [sem0, sem1, sem2, sem3]
    bufs = [rows0, rows1, rows2, rows3]
    gathers = [
        pltpu.async_copy(wv_hbm.at[im_v.at[pl.ds(k * IDS_PAD, HB)]],
                         bufs[k], sems[k])
        for k in range(NSTAGE)
    ]

    zero = jnp.zeros((L,), jnp.float32)
    mask_reg = im_v[pl.ds(NSTAGE * IDS_PAD, BPW)]

    writes = []
    for k in range(NSTAGE):
        gathers[k].wait()
        for r in range(HB):
            @pl.when(mask_reg[k * HB + r] == 0)
            def _zero_row(k=k, r=r):
                def col_body(j, _):
                    for u in range(UNROLL):
                        bufs[k][r, pl.ds((j * UNROLL + u) * L, L)] = zero
                    return 0

                lax.fori_loop(0, CHUNKS // UNROLL, col_body, 0)

        writes.append(
            pltpu.async_copy(bufs[k],
                             out_hbm.at[pl.ds(base + k * HB, HB)], semw))
    for w in writes:
        w.wait()


def kernel(word_vectors, sent_rep_token_ids, sent_rep_mask):
    wv2d = word_vectors.reshape(B * S, D)
    ids_pad = jnp.pad(
        sent_rep_token_ids.reshape(NW, NSTAGE, HB),
        ((0, 0), (0, 0), (0, IDS_PAD - HB)))
    im = jnp.concatenate(
        [ids_pad.reshape(NW, NSTAGE * IDS_PAD),
         sent_rep_mask.reshape(NW, BPW)], axis=1).reshape(-1)
    out = _gather_pool(wv2d, im)
    return out.reshape(B, N, D), sent_rep_mask


# speedup vs baseline: 22.6361x; 1.0042x over previous
import functools

import jax
import jax.numpy as jnp
from jax import lax
from jax.experimental import pallas as pl
from jax.experimental.pallas import tpu as pltpu
from jax.experimental.pallas import tpu_sc as plsc

B, S, D = 4, 4096, 2048
N = 128
TOTAL = B * N
L = 16
NC, NS = 2, 16
NW = NC * NS
BPW = TOTAL // NW
NSTAGE = 4
HB = BPW // NSTAGE
CHUNKS = D // L
UNROLL = 8

_mesh = plsc.VectorSubcoreMesh(core_axis_name="c", subcore_axis_name="s")


@functools.partial(
    pl.kernel,
    mesh=_mesh,
    out_type=jax.ShapeDtypeStruct((TOTAL, D), jnp.float32),
    scratch_types=[
        pltpu.VMEM((BPW,), jnp.int32),
        pltpu.VMEM((BPW,), jnp.int32),
        pltpu.VMEM((NSTAGE * L,), jnp.int32),
        pltpu.VMEM((HB, D), jnp.float32),
        pltpu.VMEM((HB, D), jnp.float32),
        pltpu.VMEM((HB, D), jnp.float32),
        pltpu.VMEM((HB, D), jnp.float32),
        pltpu.SemaphoreType.DMA,
        pltpu.SemaphoreType.DMA,
        pltpu.SemaphoreType.DMA,
        pltpu.SemaphoreType.DMA,
        pltpu.SemaphoreType.DMA,
        pltpu.SemaphoreType.DMA,
        pltpu.SemaphoreType.DMA,
    ],
)
def _gather_pool(wv_hbm, ids_hbm, mask_hbm, out_hbm,
                 idx_v, mask_v, stage_v, rows0, rows1, rows2, rows3,
                 semi, semm, sem0, sem1, sem2, sem3, semw):
    wid = lax.axis_index("s") * NC + lax.axis_index("c")
    base = wid * BPW
    ci = pltpu.async_copy(ids_hbm.at[pl.ds(base, BPW)], idx_v, semi)
    cm = pltpu.async_copy(mask_hbm.at[pl.ds(base, BPW)], mask_v, semm)
    ci.wait()
    boff = (base // N) * S
    idx_reg = idx_v[...] + boff
    lanes = lax.iota(jnp.int32, L)
    for k in range(NSTAGE):
        vals = idx_reg.at[(lanes + k * HB) & (L - 1)].get(
            mode="promise_in_bounds")
        stage_v[pl.ds(k * L, L)] = vals

    sems = [sem0, sem1, sem2, sem3]
    bufs = [rows0, rows1, rows2, rows3]
    gathers = [
        pltpu.async_copy(wv_hbm.at[stage_v.at[pl.ds(k * L, HB)]],
                         bufs[k], sems[k])
        for k in range(NSTAGE)
    ]

    zero = jnp.zeros((L,), jnp.float32)
    cm.wait()
    mask_reg = mask_v[...]

    writes = []
    for k in range(NSTAGE):
        gathers[k].wait()
        for r in range(HB):
            @pl.when(mask_reg[k * HB + r] == 0)
            def _zero_row(k=k, r=r):
                def col_body(j, _):
                    for u in range(UNROLL):
                        bufs[k][r, pl.ds((j * UNROLL + u) * L, L)] = zero
                    return 0

                lax.fori_loop(0, CHUNKS // UNROLL, col_body, 0)

        writes.append(
            pltpu.async_copy(bufs[k],
                             out_hbm.at[pl.ds(base + k * HB, HB)], semw))
    for w in writes:
        w.wait()


def kernel(word_vectors, sent_rep_token_ids, sent_rep_mask):
    wv2d = word_vectors.reshape(B * S, D)
    ids = sent_rep_token_ids.reshape(TOTAL)
    msk = sent_rep_mask.reshape(TOTAL)
    out = _gather_pool(wv2d, ids, msk)
    return out.reshape(B, N, D), sent_rep_mask
